# Initial kernel scaffold; baseline (speedup 1.0000x reference)
import jax
import jax.numpy as jnp
from jax.experimental import pallas as pl


def kernel(basic_block, edge_index, hidden, W_gcn, b_gcn, W_ih, W_hh, b_ih, b_hh):
    raise NotImplementedError("write your pallas kernel here")


# speedup vs baseline: 10.7171x; 10.7171x over previous
import functools

import jax
import jax.numpy as jnp
from jax import lax
from jax.experimental import pallas as pl
from jax.experimental.pallas import tpu as pltpu
from jax.experimental.pallas import tpu_sc as plsc

NC = 2
NS = 16
NW = NC * NS
LANES = 16
K_EDGE = 80

_HI = jax.lax.Precision.HIGHEST


def _xw_body(x_ref, wt_ref, o_ref):
    o_ref[...] = jnp.dot(x_ref[...], wt_ref[...], preferred_element_type=jnp.float32,
                         precision=_HI)


def _scale_body(degp_ref, xw_ref, dinv_ref, y_ref):
    cnt = jnp.sum(degp_ref[...], axis=0)
    dinv = lax.rsqrt(cnt + 2.0)
    dinv_ref[...] = dinv[:, None]
    y_ref[...] = dinv[:, None] * xw_ref[...]


def _gi_body(acc_ref, y_ref, dinv_ref, bgcn_ref, wihT_ref, bih_ref, gi_ref):
    xs = dinv_ref[...] * (acc_ref[0] + acc_ref[1] + 2.0 * y_ref[...]) + bgcn_ref[...]
    gi_ref[...] = jnp.dot(xs, wihT_ref[...], preferred_element_type=jnp.float32,
                          precision=_HI) + bih_ref[...]


def _gru_body(gi_ref, h0_ref, whhT_ref, bhh_ref, out_ref, hT_ref, hs_ref, *, bt, h):
    @pl.when(pl.program_id(0) == 0)
    def _():
        hs_ref[...] = h0_ref[...]

    whhT = whhT_ref[...]
    bhh = bhh_ref[...]

    def step(t, hv):
        g = gi_ref[pl.ds(t, 1), :]
        gh = jnp.dot(hv, whhT, preferred_element_type=jnp.float32,
                     precision=_HI) + bhh
        r = jax.nn.sigmoid(g[:, :h] + gh[:, :h])
        z = jax.nn.sigmoid(g[:, h:2 * h] + gh[:, h:2 * h])
        n = jnp.tanh(g[:, 2 * h:] + r * gh[:, 2 * h:])
        hv = (1.0 - z) * n + z * hv
        out_ref[pl.ds(t, 1), :] = hv
        return hv

    hv = lax.fori_loop(0, bt, step, hs_ref[...])
    hs_ref[...] = hv
    hT_ref[...] = hv


def _deg_partials(col, n, e):
    epw = e // NW
    mesh = plsc.VectorSubcoreMesh(core_axis_name="c", subcore_axis_name="s")

    @functools.partial(
        pl.kernel, mesh=mesh,
        out_type=jax.ShapeDtypeStruct((NW, n), jnp.float32),
        scratch_types=[
            pltpu.VMEM((epw,), jnp.int32),
            pltpu.VMEM((n,), jnp.float32),
        ],
        compiler_params=pltpu.CompilerParams(needs_layout_passes=False),
    )
    def k(col_hbm, out_hbm, col_v, deg_v):
        c = lax.axis_index("c")
        s = lax.axis_index("s")
        wid = s * NC + c
        pltpu.sync_copy(col_hbm.at[pl.ds(wid * epw, epw)], col_v)

        zz = jnp.zeros((LANES,), jnp.float32)

        def zbody(i, carry):
            deg_v[pl.ds(i * LANES, LANES)] = zz
            return carry

        lax.fori_loop(0, n // LANES, zbody, 0)

        ones = jnp.ones((LANES,), jnp.float32)

        def ebody(i, carry):
            idx = col_v[pl.ds(i * LANES, LANES)]
            plsc.addupdate_scatter(deg_v, [idx], ones)
            return carry

        lax.fori_loop(0, epw // LANES, ebody, 0)
        pltpu.sync_copy(deg_v, out_hbm.at[wid])

    return k(col)


def _msg_partials(y, er, zeros_nh, n, h, nb):
    mesh = plsc.VectorSubcoreMesh(core_axis_name="c", subcore_axis_name="s")

    @functools.partial(
        pl.kernel, mesh=mesh,
        out_type=jax.ShapeDtypeStruct((NC, n, h), jnp.float32),
        scratch_types=[
            pltpu.VMEM((nb, K_EDGE), jnp.int32),
            pltpu.VMEM((nb, K_EDGE), jnp.int32),
            pltpu.VMEM((K_EDGE, h), jnp.float32),
            pltpu.SemaphoreType.DMA,
            pltpu.VMEM_SHARED((n, h), jnp.float32),
        ],
        compiler_params=pltpu.CompilerParams(needs_layout_passes=False),
    )
    def k(y_hbm, er_hbm, z_hbm, out_hbm, rows_v, cols_v, buf_v, sem, acc_sh):
        c = lax.axis_index("c")
        s = lax.axis_index("s")
        wid = s * NC + c
        pltpu.sync_copy(er_hbm.at[0, wid], rows_v)
        pltpu.sync_copy(er_hbm.at[1, wid], cols_v)

        @pl.when(s == 0)
        def _():
            pltpu.sync_copy(z_hbm, acc_sh)

        plsc.subcore_barrier()

        def body(j, carry):
            pltpu.async_copy(y_hbm.at[rows_v.at[j]], buf_v, sem).wait()
            pltpu.sync_copy(buf_v, acc_sh.at[cols_v.at[j]], add=True)
            return carry

        lax.fori_loop(0, nb, body, 0)
        plsc.subcore_barrier()

        @pl.when(s == 0)
        def _():
            pltpu.sync_copy(acc_sh, out_hbm.at[c])

    return k(y, er, zeros_nh)


def kernel(basic_block, edge_index, hidden, W_gcn, b_gcn, W_ih, W_hh, b_ih, b_hh):
    n, d = basic_block.shape
    h = W_gcn.shape[0]
    e = edge_index.shape[1]
    epw = e // NW
    nb = epw // K_EDGE
    assert e % NW == 0 and epw % K_EDGE == 0 and n % LANES == 0

    bn = 1000
    grid_n = n // bn

    xw = pl.pallas_call(
        _xw_body,
        grid=(grid_n,),
        in_specs=[pl.BlockSpec((bn, d), lambda i: (i, 0)),
                  pl.BlockSpec((d, h), lambda i: (0, 0))],
        out_specs=pl.BlockSpec((bn, h), lambda i: (i, 0)),
        out_shape=jax.ShapeDtypeStruct((n, h), jnp.float32),
    )(basic_block, W_gcn.T)

    degp = _deg_partials(edge_index[1], n, e)

    dinv, y = pl.pallas_call(
        _scale_body,
        out_shape=[jax.ShapeDtypeStruct((n, 1), jnp.float32),
                   jax.ShapeDtypeStruct((n, h), jnp.float32)],
    )(degp, xw)

    er = edge_index.reshape(2, NW, nb, K_EDGE)
    acc = _msg_partials(y, er, jnp.zeros((n, h), jnp.float32), n, h, nb)

    gi = pl.pallas_call(
        _gi_body,
        grid=(grid_n,),
        in_specs=[pl.BlockSpec((NC, bn, h), lambda i: (0, i, 0)),
                  pl.BlockSpec((bn, h), lambda i: (i, 0)),
                  pl.BlockSpec((bn, 1), lambda i: (i, 0)),
                  pl.BlockSpec((1, h), lambda i: (0, 0)),
                  pl.BlockSpec((h, 3 * h), lambda i: (0, 0)),
                  pl.BlockSpec((1, 3 * h), lambda i: (0, 0))],
        out_specs=pl.BlockSpec((bn, 3 * h), lambda i: (i, 0)),
        out_shape=jax.ShapeDtypeStruct((n, 3 * h), jnp.float32),
    )(acc, y, dinv, b_gcn.reshape(1, h), W_ih.T, b_ih.reshape(1, 3 * h))

    bt = 1000
    outs, hT = pl.pallas_call(
        functools.partial(_gru_body, bt=bt, h=h),
        grid=(n // bt,),
        in_specs=[pl.BlockSpec((bt, 3 * h), lambda i: (i, 0)),
                  pl.BlockSpec((1, h), lambda i: (0, 0)),
                  pl.BlockSpec((h, 3 * h), lambda i: (0, 0)),
                  pl.BlockSpec((1, 3 * h), lambda i: (0, 0))],
        out_specs=[pl.BlockSpec((bt, h), lambda i: (i, 0)),
                   pl.BlockSpec((1, h), lambda i: (0, 0))],
        out_shape=[jax.ShapeDtypeStruct((n, h), jnp.float32),
                   jax.ShapeDtypeStruct((1, h), jnp.float32)],
        scratch_shapes=[pltpu.VMEM((1, h), jnp.float32)],
    )(gi, hidden, W_hh.T, b_hh.reshape(1, 3 * h))

    return outs, hT


# speedup vs baseline: 17.9076x; 1.6709x over previous
import functools

import jax
import jax.numpy as jnp
from jax import lax
from jax.experimental import pallas as pl
from jax.experimental.pallas import tpu as pltpu
from jax.experimental.pallas import tpu_sc as plsc

NC = 2
NS = 16
NW = NC * NS
LANES = 16
K_EDGE = 80

_HI = jax.lax.Precision.HIGHEST


def _xw_body(x_ref, wt_ref, o_ref):
    o_ref[...] = jnp.dot(x_ref[...], wt_ref[...], preferred_element_type=jnp.float32,
                         precision=_HI)


def _scale_body(degp_ref, xw_ref, dinv_ref, y_ref):
    cnt = jnp.sum(degp_ref[...], axis=0)
    dinv = lax.rsqrt(cnt + 2.0)
    dinv_ref[...] = dinv[:, None]
    y_ref[...] = dinv[:, None] * xw_ref[...]


def _gi_body(acc_ref, y_ref, dinv_ref, bgcn_ref, wihT_ref, bih_ref, gi_ref):
    xs = dinv_ref[...] * (acc_ref[0] + acc_ref[1] + 2.0 * y_ref[...]) + bgcn_ref[...]
    gi_ref[...] = jnp.dot(xs, wihT_ref[...], preferred_element_type=jnp.float32,
                          precision=_HI) + bih_ref[...]


def _gru_body(gi_ref, h0_ref, whhT_ref, bhh_ref, out_ref, hT_ref, hs_ref, *, bt, h):
    @pl.when(pl.program_id(0) == 0)
    def _():
        hs_ref[...] = h0_ref[...]

    whhT = whhT_ref[...]
    bhh = bhh_ref[...]

    def step(t, hv):
        g = gi_ref[pl.ds(t, 1), :]
        gh = jnp.dot(hv, whhT, preferred_element_type=jnp.float32) + bhh
        r = jax.nn.sigmoid(g[:, :h] + gh[:, :h])
        z = jax.nn.sigmoid(g[:, h:2 * h] + gh[:, h:2 * h])
        n = jnp.tanh(g[:, 2 * h:] + r * gh[:, 2 * h:])
        hv = n + z * (hv - n)
        out_ref[pl.ds(t, 1), :] = hv
        return hv

    hv = lax.fori_loop(0, bt, step, hs_ref[...], unroll=4)
    hs_ref[...] = hv
    hT_ref[...] = hv


def _deg_partials(col, n, e):
    epw = e // NW
    mesh = plsc.VectorSubcoreMesh(core_axis_name="c", subcore_axis_name="s")

    @functools.partial(
        pl.kernel, mesh=mesh,
        out_type=jax.ShapeDtypeStruct((NW, n), jnp.float32),
        scratch_types=[
            pltpu.VMEM((epw,), jnp.int32),
            pltpu.VMEM((n,), jnp.float32),
        ],
        compiler_params=pltpu.CompilerParams(needs_layout_passes=False),
    )
    def k(col_hbm, out_hbm, col_v, deg_v):
        c = lax.axis_index("c")
        s = lax.axis_index("s")
        wid = s * NC + c
        pltpu.sync_copy(col_hbm.at[pl.ds(wid * epw, epw)], col_v)

        zz = jnp.zeros((LANES,), jnp.float32)

        def zbody(i, carry):
            deg_v[pl.ds(i * LANES, LANES)] = zz
            return carry

        lax.fori_loop(0, n // LANES, zbody, 0)

        ones = jnp.ones((LANES,), jnp.float32)

        def ebody(i, carry):
            idx = col_v[pl.ds(i * LANES, LANES)]
            plsc.addupdate_scatter(deg_v, [idx], ones)
            return carry

        lax.fori_loop(0, epw // LANES, ebody, 0)
        pltpu.sync_copy(deg_v, out_hbm.at[wid])

    return k(col)


def _msg_partials(y, er, zeros_nh, n, h, nb):
    mesh = plsc.VectorSubcoreMesh(core_axis_name="c", subcore_axis_name="s")

    @functools.partial(
        pl.kernel, mesh=mesh,
        out_type=jax.ShapeDtypeStruct((NC, n, h), jnp.float32),
        scratch_types=[
            pltpu.VMEM((nb, K_EDGE), jnp.int32),
            pltpu.VMEM((nb, K_EDGE), jnp.int32),
            pltpu.VMEM((K_EDGE, h), jnp.float32),
            pltpu.SemaphoreType.DMA,
            pltpu.VMEM_SHARED((n, h), jnp.float32),
        ],
        compiler_params=pltpu.CompilerParams(needs_layout_passes=False),
    )
    def k(y_hbm, er_hbm, z_hbm, out_hbm, rows_v, cols_v, buf_v, sem, acc_sh):
        c = lax.axis_index("c")
        s = lax.axis_index("s")
        wid = s * NC + c
        pltpu.sync_copy(er_hbm.at[0, wid], rows_v)
        pltpu.sync_copy(er_hbm.at[1, wid], cols_v)

        @pl.when(s == 0)
        def _():
            pltpu.sync_copy(z_hbm, acc_sh)

        plsc.subcore_barrier()

        def body(j, carry):
            pltpu.async_copy(y_hbm.at[rows_v.at[j]], buf_v, sem).wait()
            pltpu.sync_copy(buf_v, acc_sh.at[cols_v.at[j]], add=True)
            return carry

        lax.fori_loop(0, nb, body, 0)
        plsc.subcore_barrier()

        @pl.when(s == 0)
        def _():
            pltpu.sync_copy(acc_sh, out_hbm.at[c])

    return k(y, er, zeros_nh)


def kernel(basic_block, edge_index, hidden, W_gcn, b_gcn, W_ih, W_hh, b_ih, b_hh):
    n, d = basic_block.shape
    h = W_gcn.shape[0]
    e = edge_index.shape[1]
    epw = e // NW
    nb = epw // K_EDGE
    assert e % NW == 0 and epw % K_EDGE == 0 and n % LANES == 0

    bn = 1000
    grid_n = n // bn

    xw = pl.pallas_call(
        _xw_body,
        grid=(grid_n,),
        in_specs=[pl.BlockSpec((bn, d), lambda i: (i, 0)),
                  pl.BlockSpec((d, h), lambda i: (0, 0))],
        out_specs=pl.BlockSpec((bn, h), lambda i: (i, 0)),
        out_shape=jax.ShapeDtypeStruct((n, h), jnp.float32),
    )(basic_block, W_gcn.T)

    degp = _deg_partials(edge_index[1], n, e)

    dinv, y = pl.pallas_call(
        _scale_body,
        out_shape=[jax.ShapeDtypeStruct((n, 1), jnp.float32),
                   jax.ShapeDtypeStruct((n, h), jnp.float32)],
    )(degp, xw)

    er = edge_index.reshape(2, NW, nb, K_EDGE)
    acc = _msg_partials(y, er, jnp.zeros((n, h), jnp.float32), n, h, nb)

    gi = pl.pallas_call(
        _gi_body,
        grid=(grid_n,),
        in_specs=[pl.BlockSpec((NC, bn, h), lambda i: (0, i, 0)),
                  pl.BlockSpec((bn, h), lambda i: (i, 0)),
                  pl.BlockSpec((bn, 1), lambda i: (i, 0)),
                  pl.BlockSpec((1, h), lambda i: (0, 0)),
                  pl.BlockSpec((h, 3 * h), lambda i: (0, 0)),
                  pl.BlockSpec((1, 3 * h), lambda i: (0, 0))],
        out_specs=pl.BlockSpec((bn, 3 * h), lambda i: (i, 0)),
        out_shape=jax.ShapeDtypeStruct((n, 3 * h), jnp.float32),
    )(acc, y, dinv, b_gcn.reshape(1, h), W_ih.T, b_ih.reshape(1, 3 * h))

    bt = 1000
    outs, hT = pl.pallas_call(
        functools.partial(_gru_body, bt=bt, h=h),
        grid=(n // bt,),
        in_specs=[pl.BlockSpec((bt, 3 * h), lambda i: (i, 0)),
                  pl.BlockSpec((1, h), lambda i: (0, 0)),
                  pl.BlockSpec((h, 3 * h), lambda i: (0, 0)),
                  pl.BlockSpec((1, 3 * h), lambda i: (0, 0))],
        out_specs=[pl.BlockSpec((bt, h), lambda i: (i, 0)),
                   pl.BlockSpec((1, h), lambda i: (0, 0))],
        out_shape=[jax.ShapeDtypeStruct((n, h), jnp.float32),
                   jax.ShapeDtypeStruct((1, h), jnp.float32)],
        scratch_shapes=[pltpu.VMEM((1, h), jnp.float32)],
    )(gi, hidden, W_hh.T, b_hh.reshape(1, 3 * h))

    return outs, hT


# speedup vs baseline: 21.1979x; 1.1837x over previous
import functools

import jax
import jax.numpy as jnp
from jax import lax
from jax.experimental import pallas as pl
from jax.experimental.pallas import tpu as pltpu
from jax.experimental.pallas import tpu_sc as plsc

NC = 2
NS = 16
NW = NC * NS
LANES = 16
K_EDGE = 80

_HI = jax.lax.Precision.HIGHEST


def _xw_body(x_ref, wt_ref, o_ref):
    o_ref[...] = jnp.dot(x_ref[...], wt_ref[...], preferred_element_type=jnp.float32,
                         precision=_HI)


def _scale_body(degp_ref, xw_ref, dinv_ref, y_ref):
    cnt = jnp.sum(degp_ref[...], axis=0)
    dinv = lax.rsqrt(cnt + 2.0)
    dinv_ref[...] = dinv[:, None]
    y_ref[...] = dinv[:, None] * xw_ref[...]


def _gi_body(acc_ref, y_ref, dinv_ref, bgcn_ref, wihT_ref, bih_ref, gi_ref):
    xs = dinv_ref[...] * (acc_ref[0] + acc_ref[1] + 2.0 * y_ref[...]) + bgcn_ref[...]
    gi_ref[...] = jnp.dot(xs, wihT_ref[...], preferred_element_type=jnp.float32,
                          precision=_HI) + bih_ref[...]


def _gru_body(gi_ref, h0_ref, whhT_ref, bhh_ref, out_ref, hT_ref, hs_ref, *, bt, h):
    @pl.when(pl.program_id(0) == 0)
    def _():
        hs_ref[...] = h0_ref[...]

    whhT = whhT_ref[...]
    bhh = bhh_ref[...]

    def sig(x):
        return 0.5 + 0.5 * jnp.tanh(0.5 * x)

    def step(t, hv):
        gb = gi_ref[pl.ds(t, 1), :]
        hcol = hv.reshape(h, 1)
        p = whhT * hcol
        p = p[:h // 2] + p[h // 2:]
        p = p[:h // 4] + p[h // 4:]
        p = p[:h // 8] + p[h // 8:]
        p = p[:h // 16] + p[h // 16:]
        gh = jnp.sum(p, axis=0, keepdims=True) + bhh
        r = sig(gb[:, :h] + gh[:, :h])
        z = sig(gb[:, h:2 * h] + gh[:, h:2 * h])
        n = jnp.tanh(gb[:, 2 * h:] + r * gh[:, 2 * h:])
        hv = n + z * (hv - n)
        out_ref[pl.ds(t, 1), :] = hv
        return hv

    hv = lax.fori_loop(0, bt, step, hs_ref[...], unroll=4)
    hs_ref[...] = hv
    hT_ref[...] = hv


def _deg_partials(col, n, e):
    epw = e // NW
    mesh = plsc.VectorSubcoreMesh(core_axis_name="c", subcore_axis_name="s")

    @functools.partial(
        pl.kernel, mesh=mesh,
        out_type=jax.ShapeDtypeStruct((NW, n), jnp.float32),
        scratch_types=[
            pltpu.VMEM((epw,), jnp.int32),
            pltpu.VMEM((n,), jnp.float32),
        ],
        compiler_params=pltpu.CompilerParams(needs_layout_passes=False),
    )
    def k(col_hbm, out_hbm, col_v, deg_v):
        c = lax.axis_index("c")
        s = lax.axis_index("s")
        wid = s * NC + c
        pltpu.sync_copy(col_hbm.at[pl.ds(wid * epw, epw)], col_v)

        zz = jnp.zeros((LANES,), jnp.float32)

        def zbody(i, carry):
            deg_v[pl.ds(i * LANES, LANES)] = zz
            return carry

        lax.fori_loop(0, n // LANES, zbody, 0)

        ones = jnp.ones((LANES,), jnp.float32)

        def ebody(i, carry):
            idx = col_v[pl.ds(i * LANES, LANES)]
            plsc.addupdate_scatter(deg_v, [idx], ones)
            return carry

        lax.fori_loop(0, epw // LANES, ebody, 0)
        pltpu.sync_copy(deg_v, out_hbm.at[wid])

    return k(col)


def _msg_partials(y, er, zeros_nh, n, h, nb):
    mesh = plsc.VectorSubcoreMesh(core_axis_name="c", subcore_axis_name="s")

    @functools.partial(
        pl.kernel, mesh=mesh,
        out_type=jax.ShapeDtypeStruct((NC, n, h), jnp.float32),
        scratch_types=[
            pltpu.VMEM((nb, K_EDGE), jnp.int32),
            pltpu.VMEM((nb, K_EDGE), jnp.int32),
            pltpu.VMEM((K_EDGE, h), jnp.float32),
            pltpu.SemaphoreType.DMA,
            pltpu.VMEM_SHARED((n, h), jnp.float32),
        ],
        compiler_params=pltpu.CompilerParams(needs_layout_passes=False),
    )
    def k(y_hbm, er_hbm, z_hbm, out_hbm, rows_v, cols_v, buf_v, sem, acc_sh):
        c = lax.axis_index("c")
        s = lax.axis_index("s")
        wid = s * NC + c
        pltpu.sync_copy(er_hbm.at[0, wid], rows_v)
        pltpu.sync_copy(er_hbm.at[1, wid], cols_v)

        @pl.when(s == 0)
        def _():
            pltpu.sync_copy(z_hbm, acc_sh)

        plsc.subcore_barrier()

        def body(j, carry):
            pltpu.async_copy(y_hbm.at[rows_v.at[j]], buf_v, sem).wait()
            pltpu.sync_copy(buf_v, acc_sh.at[cols_v.at[j]], add=True)
            return carry

        lax.fori_loop(0, nb, body, 0)
        plsc.subcore_barrier()

        @pl.when(s == 0)
        def _():
            pltpu.sync_copy(acc_sh, out_hbm.at[c])

    return k(y, er, zeros_nh)


def kernel(basic_block, edge_index, hidden, W_gcn, b_gcn, W_ih, W_hh, b_ih, b_hh):
    n, d = basic_block.shape
    h = W_gcn.shape[0]
    e = edge_index.shape[1]
    epw = e // NW
    nb = epw // K_EDGE
    assert e % NW == 0 and epw % K_EDGE == 0 and n % LANES == 0

    bn = 1000
    grid_n = n // bn

    xw = pl.pallas_call(
        _xw_body,
        grid=(grid_n,),
        in_specs=[pl.BlockSpec((bn, d), lambda i: (i, 0)),
                  pl.BlockSpec((d, h), lambda i: (0, 0))],
        out_specs=pl.BlockSpec((bn, h), lambda i: (i, 0)),
        out_shape=jax.ShapeDtypeStruct((n, h), jnp.float32),
    )(basic_block, W_gcn.T)

    degp = _deg_partials(edge_index[1], n, e)

    dinv, y = pl.pallas_call(
        _scale_body,
        out_shape=[jax.ShapeDtypeStruct((n, 1), jnp.float32),
                   jax.ShapeDtypeStruct((n, h), jnp.float32)],
    )(degp, xw)

    er = edge_index.reshape(2, NW, nb, K_EDGE)
    acc = _msg_partials(y, er, jnp.zeros((n, h), jnp.float32), n, h, nb)

    gi = pl.pallas_call(
        _gi_body,
        grid=(grid_n,),
        in_specs=[pl.BlockSpec((NC, bn, h), lambda i: (0, i, 0)),
                  pl.BlockSpec((bn, h), lambda i: (i, 0)),
                  pl.BlockSpec((bn, 1), lambda i: (i, 0)),
                  pl.BlockSpec((1, h), lambda i: (0, 0)),
                  pl.BlockSpec((h, 3 * h), lambda i: (0, 0)),
                  pl.BlockSpec((1, 3 * h), lambda i: (0, 0))],
        out_specs=pl.BlockSpec((bn, 3 * h), lambda i: (i, 0)),
        out_shape=jax.ShapeDtypeStruct((n, 3 * h), jnp.float32),
    )(acc, y, dinv, b_gcn.reshape(1, h), W_ih.T, b_ih.reshape(1, 3 * h))

    bt = 1000
    outs, hT = pl.pallas_call(
        functools.partial(_gru_body, bt=bt, h=h),
        grid=(n // bt,),
        in_specs=[pl.BlockSpec((bt, 3 * h), lambda i: (i, 0)),
                  pl.BlockSpec((1, h), lambda i: (0, 0)),
                  pl.BlockSpec((h, 3 * h), lambda i: (0, 0)),
                  pl.BlockSpec((1, 3 * h), lambda i: (0, 0))],
        out_specs=[pl.BlockSpec((bt, h), lambda i: (i, 0)),
                   pl.BlockSpec((1, h), lambda i: (0, 0))],
        out_shape=[jax.ShapeDtypeStruct((n, h), jnp.float32),
                   jax.ShapeDtypeStruct((1, h), jnp.float32)],
        scratch_shapes=[pltpu.VMEM((1, h), jnp.float32)],
    )(gi, hidden, W_hh.T, b_hh.reshape(1, 3 * h))

    return outs, hT
